# baseline (speedup 1.0000x reference)
import functools

import jax
import jax.numpy as jnp
from jax import lax
from jax.experimental import pallas as pl
from jax.experimental.pallas import tpu as pltpu
from jax.experimental.pallas import tpu_sc as plsc

EMBED_DIM = 64
ROW_PAD = 128
BATCH = 16384

NC = 2
NS = 16
L = 16
NW = NC * NS
B_PER_W = BATCH // NW
CHUNK = 64
N_CHUNKS = B_PER_W // CHUNK
GROUPS = CHUNK // L


def _mf_kernel(u_hbm, i_hbm, uw_hbm, iw_hbm, out_hbm,
               uidx_v, iidx_v, ue_v, ie_v, out_v, sems):
    wid = lax.axis_index("s") * NC + lax.axis_index("c")
    base = pl.multiple_of(wid * B_PER_W, B_PER_W)

    pltpu.sync_copy(u_hbm.at[pl.ds(base, B_PER_W)], uidx_v)
    pltpu.sync_copy(i_hbm.at[pl.ds(base, B_PER_W)], iidx_v)

    def gather_chunk(c, slot):
        sl = pl.ds(c * CHUNK, CHUNK)
        pltpu.async_copy(uw_hbm.at[uidx_v.at[sl]], ue_v.at[slot],
                         sems.at[slot])
        pltpu.async_copy(iw_hbm.at[iidx_v.at[sl]], ie_v.at[slot],
                         sems.at[slot])

    def wait_chunk(slot):
        zeros = jnp.zeros((CHUNK,), jnp.int32)
        pltpu.make_async_copy(
            uw_hbm.at[zeros], ue_v.at[slot], sems.at[slot]).wait()
        pltpu.make_async_copy(
            iw_hbm.at[zeros], ie_v.at[slot], sems.at[slot]).wait()

    iota = lax.iota(jnp.int32, L)
    ones = jnp.ones((L,), jnp.int32)

    def compute_chunk(c, slot):
        ue_w = ue_v
        ie_w = ie_v
        himask = jnp.full((L,), -65536, jnp.int32)
        sixteen = jnp.full((L,), 16, jnp.int32)
        for g in range(GROUPS):
            rows = jnp.full((L,), g * L, jnp.int32) + iota
            col = jnp.zeros((L,), jnp.int32)
            accs = [jnp.zeros((L,), jnp.float32) for _ in range(4)]
            for w in range(EMBED_DIM // 2):
                aw = plsc.load_gather(ue_w.at[slot], [rows, col])
                bw = plsc.load_gather(ie_w.at[slot], [rows, col])
                alo = lax.bitcast_convert_type(
                    lax.shift_left(aw, sixteen), jnp.float32)
                blo = lax.bitcast_convert_type(
                    lax.shift_left(bw, sixteen), jnp.float32)
                ahi = lax.bitcast_convert_type(aw & himask, jnp.float32)
                bhi = lax.bitcast_convert_type(bw & himask, jnp.float32)
                accs[w % 4] = accs[w % 4] + alo * blo + ahi * bhi
                if w != EMBED_DIM // 2 - 1:
                    col = col + ones
            out_v[pl.ds(c * CHUNK + g * L, L)] = (
                (accs[0] + accs[1]) + (accs[2] + accs[3]))

    gather_chunk(0, 0)

    def body(j, carry):
        c0 = j * 2
        wait_chunk(0)
        gather_chunk(c0 + 1, 1)
        compute_chunk(c0, 0)
        wait_chunk(1)

        @pl.when(c0 + 2 < N_CHUNKS)
        def _prefetch():
            gather_chunk(c0 + 2, 0)

        compute_chunk(c0 + 1, 1)
        return carry

    lax.fori_loop(0, N_CHUNKS // 2, body, 0, unroll=False)

    pltpu.sync_copy(out_v, out_hbm.at[pl.ds(base, B_PER_W)])


@jax.jit
def kernel(u, i, user_w, item_w):
    def pack(w):
        n = w.shape[0]
        wp = jnp.pad(w, ((0, 0), (0, ROW_PAD - EMBED_DIM)))
        wb = wp.astype(jnp.bfloat16).reshape(n, ROW_PAD // 2, 2)
        return lax.bitcast_convert_type(wb, jnp.int32)

    uw_pad = pack(user_w)
    iw_pad = pack(item_w)
    mesh = plsc.VectorSubcoreMesh(core_axis_name="c", subcore_axis_name="s")
    run = functools.partial(
        pl.kernel, mesh=mesh,
        compiler_params=pltpu.CompilerParams(
            use_tc_tiling_on_sc=False, needs_layout_passes=False),
        out_type=jax.ShapeDtypeStruct((BATCH,), jnp.float32),
        scratch_types=[
            pltpu.VMEM((B_PER_W,), jnp.int32),
            pltpu.VMEM((B_PER_W,), jnp.int32),
            pltpu.VMEM((2, CHUNK, ROW_PAD // 2), jnp.int32),
            pltpu.VMEM((2, CHUNK, ROW_PAD // 2), jnp.int32),
            pltpu.VMEM((B_PER_W,), jnp.float32),
            pltpu.SemaphoreType.DMA((2,)),
        ],
    )(_mf_kernel)
    return run(u.astype(jnp.int32), i.astype(jnp.int32), uw_pad, iw_pad)


# speedup vs baseline: 5.5604x; 5.5604x over previous
import functools

import jax
import jax.numpy as jnp
from jax import lax
from jax.experimental import pallas as pl
from jax.experimental.pallas import tpu as pltpu
from jax.experimental.pallas import tpu_sc as plsc

EMBED_DIM = 64
ROW_PAD = 128
BATCH = 16384

NC = 2
NS = 16
L = 16
NW = NC * NS
B_PER_W = BATCH // NW
CHUNK = 64
N_CHUNKS = B_PER_W // CHUNK
GROUPS = CHUNK // L


def _mf_kernel(u_hbm, i_hbm, uw_hbm, iw_hbm, out_hbm,
               uidx_v, iidx_v, ue_v, ie_v, out_v, sems):
    wid = lax.axis_index("s") * NC + lax.axis_index("c")
    base = pl.multiple_of(wid * B_PER_W, B_PER_W)

    pltpu.sync_copy(u_hbm.at[pl.ds(base, B_PER_W)], uidx_v)
    pltpu.sync_copy(i_hbm.at[pl.ds(base, B_PER_W)], iidx_v)

    def gather_chunk(c, slot):
        sl = pl.ds(c * CHUNK, CHUNK)
        pltpu.async_copy(uw_hbm.at[uidx_v.at[sl]], ue_v.at[slot],
                         sems.at[slot])
        pltpu.async_copy(iw_hbm.at[iidx_v.at[sl]], ie_v.at[slot],
                         sems.at[slot])

    def wait_chunk(slot):
        zeros = jnp.zeros((CHUNK,), jnp.int32)
        pltpu.make_async_copy(
            uw_hbm.at[zeros], ue_v.at[slot], sems.at[slot]).wait()
        pltpu.make_async_copy(
            iw_hbm.at[zeros], ie_v.at[slot], sems.at[slot]).wait()

    iota = lax.iota(jnp.int32, L)
    ones = jnp.ones((L,), jnp.int32)

    def compute_chunk(c, slot):
        for g in range(GROUPS):
            rows = jnp.full((L,), g * L, jnp.int32) + iota
            col = jnp.zeros((L,), jnp.int32)
            accs = [jnp.zeros((L,), jnp.float32) for _ in range(4)]
            for d in range(EMBED_DIM):
                a = plsc.load_gather(ue_v.at[slot], [rows, col])
                b = plsc.load_gather(ie_v.at[slot], [rows, col])
                accs[d % 4] = accs[d % 4] + a * b
                if d != EMBED_DIM - 1:
                    col = col + ones
            out_v[pl.ds(c * CHUNK + g * L, L)] = (
                (accs[0] + accs[1]) + (accs[2] + accs[3]))

    gather_chunk(0, 0)

    def body(j, carry):
        c0 = j * 2
        wait_chunk(0)
        gather_chunk(c0 + 1, 1)
        compute_chunk(c0, 0)
        wait_chunk(1)

        @pl.when(c0 + 2 < N_CHUNKS)
        def _prefetch():
            gather_chunk(c0 + 2, 0)

        compute_chunk(c0 + 1, 1)
        return carry

    lax.fori_loop(0, N_CHUNKS // 2, body, 0, unroll=False)

    pltpu.sync_copy(out_v, out_hbm.at[pl.ds(base, B_PER_W)])


@jax.jit
def kernel(u, i, user_w, item_w):
    uw_pad = jnp.pad(user_w, ((0, 0), (0, ROW_PAD - EMBED_DIM)))
    iw_pad = jnp.pad(item_w, ((0, 0), (0, ROW_PAD - EMBED_DIM)))
    mesh = plsc.VectorSubcoreMesh(core_axis_name="c", subcore_axis_name="s")
    run = functools.partial(
        pl.kernel, mesh=mesh,
        compiler_params=pltpu.CompilerParams(
            use_tc_tiling_on_sc=False, needs_layout_passes=False),
        out_type=jax.ShapeDtypeStruct((BATCH,), jnp.float32),
        scratch_types=[
            pltpu.VMEM((B_PER_W,), jnp.int32),
            pltpu.VMEM((B_PER_W,), jnp.int32),
            pltpu.VMEM((2, CHUNK, ROW_PAD), jnp.float32),
            pltpu.VMEM((2, CHUNK, ROW_PAD), jnp.float32),
            pltpu.VMEM((B_PER_W,), jnp.float32),
            pltpu.SemaphoreType.DMA((2,)),
        ],
    )(_mf_kernel)
    return run(u.astype(jnp.int32), i.astype(jnp.int32), uw_pad, iw_pad)
